# Initial kernel scaffold; baseline (speedup 1.0000x reference)
import jax
import jax.numpy as jnp
from jax.experimental import pallas as pl


def kernel(nfeatures, efeatures, params, edge_index):
    raise NotImplementedError("write your pallas kernel here")


# speedup vs baseline: 3.7388x; 3.7388x over previous
import functools

import jax
import jax.numpy as jnp
from jax import lax
from jax.experimental import pallas as pl
from jax.experimental.pallas import tpu as pltpu
from jax.experimental.pallas import tpu_sc as plsc

NN = 100000
NE = 3200000
ROW = 128
NROWS = 25600
NE_PAD = NROWS * ROW
NC, NS, NW = 2, 16, 32
RPW = NROWS // NW
RFIRE_G = 4
RFIRE_S = 8
NN_ACC = 100096
SLICE = NN_ACC // NS
TBL_SLICE = NN // NS
BLK_E = 8192
BLK_N = 4000


def _leaky(x):
    return jnp.where(x > 0, x, 0.01 * x)


def _mlp_tc(xs, w1ts, b1, w2t, b2, w3t, b3, g, b, blk, residual, psum_idx=None):
    n_rows = xs[0].shape[0]
    out_f = w3t.shape[1]
    grid = (n_rows // blk,)
    n_x = len(xs)

    def body(*refs):
        x_refs = refs[:n_x]
        o_ref = refs[-1]
        wrefs = refs[n_x:-1]
        it = iter(wrefs)
        streams = []
        for i, xr in enumerate(x_refs):
            if i == psum_idx:
                streams.append(xr[0] + xr[1])
            else:
                streams.append(xr[...])
        z = jnp.zeros((blk, w2t.shape[0]), jnp.float32)
        for xv in streams:
            w1r = next(it)
            z = z + jnp.dot(xv, w1r[...], preferred_element_type=jnp.float32)
        b1r = next(it)
        z = _leaky(z + b1r[...])
        w2r, b2r = next(it), next(it)
        z = _leaky(jnp.dot(z, w2r[...], preferred_element_type=jnp.float32) + b2r[...])
        w3r, b3r = next(it), next(it)
        f = jnp.dot(z, w3r[...], preferred_element_type=jnp.float32) + b3r[...]
        if g is not None:
            gr, br = next(it), next(it)
            mu = jnp.mean(f, axis=-1, keepdims=True)
            var = jnp.mean((f - mu) * (f - mu), axis=-1, keepdims=True)
            f = (f - mu) * lax.rsqrt(var + 1e-5) * gr[...] + br[...]
        if residual is not None:
            f = f + streams[residual]
        o_ref[...] = f

    in_specs = []
    operands = []
    for i, x in enumerate(xs):
        if i == psum_idx:
            in_specs.append(pl.BlockSpec((2, blk, 16), lambda i: (0, i, 0)))
        else:
            in_specs.append(pl.BlockSpec((blk, x.shape[1]), lambda i: (i, 0)))
        operands.append(x)
    wlist = list(w1ts) + [b1, w2t, b2, w3t, b3]
    if g is not None:
        wlist += [g, b]
    for w in wlist:
        in_specs.append(pl.BlockSpec(w.shape, lambda i, _r=len(w.shape): (0,) * _r))
        operands.append(w)
    return pl.pallas_call(
        body,
        grid=grid,
        in_specs=in_specs,
        out_specs=pl.BlockSpec((blk, out_f), lambda i: (i, 0)),
        out_shape=jax.ShapeDtypeStruct((n_rows, out_f), jnp.float32),
    )(*operands)


def _prep(p, n_split):
    wi = p['Wi']
    splits = []
    off = 0
    for f in n_split:
        splits.append(jnp.transpose(wi[:, off:off + f]))
        off += f
    b1 = p['bi'].reshape(1, -1)
    wh, bh = p['hidden'][0]
    w2t = jnp.transpose(wh)
    b2 = bh.reshape(1, -1)
    w3t = jnp.transpose(p['Wo'])
    b3 = p['bo'].reshape(1, -1)
    if 'g' in p:
        g = p['g'].reshape(1, -1)
        b = p['b'].reshape(1, -1)
    else:
        g = b = None
    return splits, b1, w2t, b2, w3t, b3, g, b


def _sc_gather(n_tbl, src2d, dst2d):
    mesh = plsc.VectorSubcoreMesh(core_axis_name="c", subcore_axis_name="s")

    @functools.partial(
        pl.kernel,
        mesh=mesh,
        compiler_params=pltpu.CompilerParams(use_tc_tiling_on_sc=False),
        out_type=(
            jax.ShapeDtypeStruct((NE_PAD, 16), jnp.float32),
            jax.ShapeDtypeStruct((NE_PAD, 16), jnp.float32),
        ),
        scratch_types=[
            pltpu.VMEM_SHARED((NN, 16), jnp.float32),
            pltpu.VMEM((RFIRE_G, ROW), jnp.int32),
            pltpu.VMEM((RFIRE_G, ROW), jnp.int32),
            pltpu.VMEM((RFIRE_G * ROW, 16), jnp.float32),
            pltpu.VMEM((RFIRE_G * ROW, 16), jnp.float32),
            pltpu.SemaphoreType.DMA,
            pltpu.SemaphoreType.DMA,
        ],
    )
    def k(tbl_hbm, src_hbm, dst_hbm, ns_hbm, nd_hbm,
          tbl_sh, idx_s, idx_d, rows_s, rows_d, sem_s, sem_d):
        c = lax.axis_index("c")
        s = lax.axis_index("s")
        wid = s * NC + c
        t0 = jnp.minimum(s * SLICE, NN - SLICE)
        pltpu.sync_copy(tbl_hbm.at[pl.ds(t0, SLICE)],
                        tbl_sh.at[pl.ds(t0, SLICE)])
        plsc.subcore_barrier()
        base_row = wid * RPW

        def outer(t, _):
            r0 = base_row + t * RFIRE_G
            pltpu.sync_copy(src_hbm.at[pl.ds(r0, RFIRE_G)], idx_s)
            pltpu.sync_copy(dst_hbm.at[pl.ds(r0, RFIRE_G)], idx_d)
            cps = [pltpu.async_copy(tbl_sh.at[idx_s.at[j]],
                                    rows_s.at[pl.ds(j * ROW, ROW)], sem_s)
                   for j in range(RFIRE_G)]
            cpd = [pltpu.async_copy(tbl_sh.at[idx_d.at[j]],
                                    rows_d.at[pl.ds(j * ROW, ROW)], sem_d)
                   for j in range(RFIRE_G)]
            for cp in cps:
                cp.wait()
            for cp in cpd:
                cp.wait()
            pltpu.sync_copy(rows_s, ns_hbm.at[pl.ds(r0 * ROW, RFIRE_G * ROW)])
            pltpu.sync_copy(rows_d, nd_hbm.at[pl.ds(r0 * ROW, RFIRE_G * ROW)])
            return 0

        lax.fori_loop(0, RPW // RFIRE_G, outer, 0)

    return k(n_tbl, src2d, dst2d)


def _sc_scatter(e_pad, dst2d_acc, zeros_slice):
    mesh = plsc.VectorSubcoreMesh(core_axis_name="c", subcore_axis_name="s")

    @functools.partial(
        pl.kernel,
        mesh=mesh,
        compiler_params=pltpu.CompilerParams(use_tc_tiling_on_sc=False),
        out_type=jax.ShapeDtypeStruct((2, NN, 16), jnp.float32),
        scratch_types=[
            pltpu.VMEM_SHARED((NN_ACC, 16), jnp.float32),
            pltpu.VMEM((RFIRE_S * ROW, 16), jnp.float32),
            pltpu.VMEM((RFIRE_S, ROW), jnp.int32),
            pltpu.SemaphoreType.DMA,
        ],
    )
    def k(e_hbm, dst_hbm, z_hbm, out_hbm, acc_sh, e_v, idx_v, sem):
        c = lax.axis_index("c")
        s = lax.axis_index("s")
        pltpu.sync_copy(z_hbm, acc_sh.at[pl.ds(s * SLICE, SLICE)])
        plsc.subcore_barrier()
        base_row = (c * NS + s) * RPW

        def outer(t, _):
            r0 = base_row + t * RFIRE_S
            pltpu.sync_copy(e_hbm.at[pl.ds(r0 * ROW, RFIRE_S * ROW)], e_v)
            pltpu.sync_copy(dst_hbm.at[pl.ds(r0, RFIRE_S)], idx_v)
            cps = [pltpu.async_copy(e_v.at[pl.ds(j * ROW, ROW)],
                                    acc_sh.at[idx_v.at[j]], sem, add=True)
                   for j in range(RFIRE_S)]
            for cp in cps:
                cp.wait()
            return 0

        lax.fori_loop(0, RPW // RFIRE_S, outer, 0)
        plsc.subcore_barrier()
        start = jnp.minimum(s * SLICE, NN - SLICE)
        pltpu.sync_copy(acc_sh.at[pl.ds(start, SLICE)],
                        out_hbm.at[c].at[pl.ds(start, SLICE)])

    return k(e_pad, dst2d_acc, zeros_slice)


def kernel(nfeatures, efeatures, params, edge_index):
    src = edge_index[0].astype(jnp.int32)
    dst = edge_index[1].astype(jnp.int32)
    pad = NE_PAD - NE
    fill = (jnp.arange(pad, dtype=jnp.int32) % NN)
    src2d = jnp.concatenate([src, fill]).reshape(NROWS, ROW)
    dst2d = jnp.concatenate([dst, fill]).reshape(NROWS, ROW)
    fill_acc = NN + (jnp.arange(pad, dtype=jnp.int32) % (NN_ACC - NN))
    dst2d_acc = jnp.concatenate([dst, fill_acc]).reshape(NROWS, ROW)
    zeros_slice = jnp.zeros((SLICE, 16), jnp.float32)

    ef_pad = jnp.pad(efeatures, ((0, pad), (0, 0)))

    pn = _prep(params['enc_n'], [16])
    n = _mlp_tc([nfeatures], pn[0], *pn[1:], blk=BLK_N, residual=None)
    pe = _prep(params['enc_e'], [4])
    e = _mlp_tc([ef_pad], pe[0], *pe[1:], blk=BLK_E, residual=None)

    for i in range(2):
        ns, nd = _sc_gather(n, src2d, dst2d)
        pp = _prep(params['proc_e'][i], [16, 16, 16])
        e = _mlp_tc([e, ns, nd], pp[0], *pp[1:], blk=BLK_E, residual=0)
        psum = _sc_scatter(e, dst2d_acc, zeros_slice)
        pq = _prep(params['proc_n'][i], [16, 16])
        n = _mlp_tc([n, psum], pq[0], *pq[1:], blk=BLK_N, residual=0,
                    psum_idx=1)

    pd = _prep(params['dec'], [16])
    return _mlp_tc([n], pd[0], *pd[1:], blk=BLK_N, residual=None)
